# baseline (speedup 1.0000x reference)
import jax
import jax.numpy as jnp
from jax.experimental import pallas as pl

_SPLITS = 2


def _est_body(*refs):
    (xall_ref, x_ref, st_ref, b_ref, sr_ref, alr_ref, temp_ref) = refs[:7]
    S = _SPLITS
    w_refs = refs[7:7 + S]
    win_refs = refs[7 + S:7 + 2 * S]
    wout_refs = refs[7 + 2 * S:7 + 3 * S]
    ns_ref, out_ref = refs[7 + 3 * S:]

    u = pl.program_id(0)
    nu = pl.num_programs(0)
    temp = temp_ref[0, 0]

    x_all = xall_ref[...]
    alr = alr_ref[...][:, :, 0]
    logits = jnp.sum(x_all * alr[:, None, :], axis=-1) / temp
    m = jnp.max(logits, axis=0)
    e = jnp.exp(logits - m[None, :])
    denom = jnp.sum(e, axis=0)
    onehot = (jax.lax.broadcasted_iota(jnp.int32, (nu, 1), 0) == u
              ).astype(jnp.float32)
    lr_u = (jnp.sum(e * onehot, axis=0) / denom)[:, None]
    sr_u = jnp.sum(sr_ref[...][:, :, 0] * onehot)

    x_u = x_ref[0]
    st_u = st_ref[0]
    sts = st_u * sr_u
    D = x_u.shape[1]
    N = st_u.shape[1]
    Kd, Kn = D // S, N // S

    feed = jnp.dot(x_u[:, :Kd], win_refs[0][0],
                   preferred_element_type=jnp.float32)
    echo = jnp.dot(sts[:, :Kn], w_refs[0][0],
                   preferred_element_type=jnp.float32)
    for i in range(1, S):
        feed += jnp.dot(x_u[:, i * Kd:(i + 1) * Kd], win_refs[i][0],
                        preferred_element_type=jnp.float32)
        echo += jnp.dot(sts[:, i * Kn:(i + 1) * Kn], w_refs[i][0],
                        preferred_element_type=jnp.float32)
    act = jnp.tanh(feed + echo + b_ref[0, 0, :][None, :])
    ns = (1.0 - lr_u) * st_u + lr_u * act
    ns_ref[...] = ns[None, :, :]
    out = jnp.dot(ns[:, :Kn], wout_refs[0][0],
                  preferred_element_type=jnp.float32)
    for i in range(1, S):
        out += jnp.dot(ns[:, i * Kn:(i + 1) * Kn], wout_refs[i][0],
                       preferred_element_type=jnp.float32)
    out_ref[...] = out[None, :, :]


def kernel(X, state, W, Win, bias, Wout, sr, adaptive_lr, temperature,
           w_h, w_o, w_d, win_h, win_o, win_d):
    B, U, D = X.shape
    N = state.shape[2]
    O = Wout.shape[2]
    S = _SPLITS
    Xt = X.transpose(1, 0, 2)
    stt = state.transpose(1, 0, 2)
    temp2 = temperature.reshape(1, 1)

    in_specs = [
        pl.BlockSpec((U, B, D), lambda u: (0, 0, 0)),
        pl.BlockSpec((1, B, D), lambda u: (u, 0, 0)),
        pl.BlockSpec((1, B, N), lambda u: (u, 0, 0)),
        pl.BlockSpec((1, 1, N), lambda u: (u, 0, 0)),
        pl.BlockSpec((U, 1, 1), lambda u: (0, 0, 0)),
        pl.BlockSpec((U, D, 1), lambda u: (0, 0, 0)),
        pl.BlockSpec((1, 1), lambda u: (0, 0)),
    ]
    args = [Xt, Xt, stt, bias, sr, adaptive_lr, temp2]
    for i in range(S):
        in_specs.append(pl.BlockSpec((1, N // S, N), lambda u, i=i: (u, i, 0)))
        args.append(W)
    for i in range(S):
        in_specs.append(pl.BlockSpec((1, D // S, N), lambda u, i=i: (u, i, 0)))
        args.append(Win)
    for i in range(S):
        in_specs.append(pl.BlockSpec((1, N // S, O), lambda u, i=i: (u, i, 0)))
        args.append(Wout)

    ns, out = pl.pallas_call(
        _est_body,
        grid=(U,),
        in_specs=in_specs,
        out_specs=[
            pl.BlockSpec((1, B, N), lambda u: (u, 0, 0)),
            pl.BlockSpec((1, B, O), lambda u: (u, 0, 0)),
        ],
        out_shape=[
            jax.ShapeDtypeStruct((U, B, N), jnp.float32),
            jax.ShapeDtypeStruct((U, B, O), jnp.float32),
        ],
    )(*args)
    return ns.transpose(1, 0, 2), out.transpose(1, 0, 2)


# speedup vs baseline: 1.0412x; 1.0412x over previous
import jax
import jax.numpy as jnp
from jax.experimental import pallas as pl


def _est_body(xflat_ref, alrbd_ref, x_ref, st_ref, w_ref, win_ref, b_ref,
              wout_ref, sr_ref, temp_ref, ns_ref, out_ref):
    u = pl.program_id(0)
    nu = pl.num_programs(0)
    temp = temp_ref[0, 0]

    logits = jnp.dot(xflat_ref[...], alrbd_ref[...],
                     preferred_element_type=jnp.float32) / temp
    lr = jax.nn.softmax(logits, axis=1)
    onehot = (jax.lax.broadcasted_iota(jnp.int32, (1, nu), 1) == u
              ).astype(jnp.float32)
    lr_u = jnp.sum(lr * onehot, axis=1)[:, None]
    sr_u = jnp.sum(sr_ref[...] * onehot)

    B = x_ref.shape[0]
    D = x_ref.shape[2] * x_ref.shape[3]
    N = st_ref.shape[2] * st_ref.shape[3]
    x_u = x_ref[:, 0].reshape(B, D)
    st_u = st_ref[:, 0].reshape(B, N)
    feed = jnp.dot(x_u, win_ref[0], preferred_element_type=jnp.float32)
    echo = jnp.dot(st_u * sr_u, w_ref[0], preferred_element_type=jnp.float32)
    act = jnp.tanh(feed + echo + b_ref[0, 0, :][None, :])
    ns = (1.0 - lr_u) * st_u + lr_u * act
    out = jnp.dot(ns, wout_ref[0], preferred_element_type=jnp.float32)
    ns_ref[...] = ns.reshape(B, 1, 8, N // 8)
    out_ref[...] = out.reshape(B, 1, 8, out.shape[1] // 8)


def kernel(X, state, W, Win, bias, Wout, sr, adaptive_lr, temperature,
           w_h, w_o, w_d, win_h, win_o, win_d):
    B, U, D = X.shape
    N = state.shape[2]
    O = Wout.shape[2]
    X_flat = X.reshape(B, U * D)
    X4 = X.reshape(B, U, 8, D // 8)
    st4 = state.reshape(B, U, 8, N // 8)
    unit_of_row = jnp.repeat(jnp.arange(U), D)
    alr_bd = (adaptive_lr.reshape(U * D)[:, None] *
              (unit_of_row[:, None] == jnp.arange(U)[None, :]))
    sr2 = sr.reshape(1, U)
    temp2 = temperature.reshape(1, 1)
    ns, out = pl.pallas_call(
        _est_body,
        grid=(U,),
        in_specs=[
            pl.BlockSpec((B, U * D), lambda u: (0, 0)),
            pl.BlockSpec((U * D, U), lambda u: (0, 0)),
            pl.BlockSpec((B, 1, 8, D // 8), lambda u: (0, u, 0, 0)),
            pl.BlockSpec((B, 1, 8, N // 8), lambda u: (0, u, 0, 0)),
            pl.BlockSpec((1, N, N), lambda u: (u, 0, 0)),
            pl.BlockSpec((1, D, N), lambda u: (u, 0, 0)),
            pl.BlockSpec((1, 1, N), lambda u: (u, 0, 0)),
            pl.BlockSpec((1, N, O), lambda u: (u, 0, 0)),
            pl.BlockSpec((1, U), lambda u: (0, 0)),
            pl.BlockSpec((1, 1), lambda u: (0, 0)),
        ],
        out_specs=[
            pl.BlockSpec((B, 1, 8, N // 8), lambda u: (0, u, 0, 0)),
            pl.BlockSpec((B, 1, 8, O // 8), lambda u: (0, u, 0, 0)),
        ],
        out_shape=[
            jax.ShapeDtypeStruct((B, U, 8, N // 8), jnp.float32),
            jax.ShapeDtypeStruct((B, U, 8, O // 8), jnp.float32),
        ],
    )(X_flat, alr_bd, X4, st4, W, Win, bias, Wout, sr2, temp2)
    return ns.reshape(B, U, N), out.reshape(B, U, O)


# speedup vs baseline: 1.0503x; 1.0087x over previous
import jax
import jax.numpy as jnp
from jax.experimental import pallas as pl


def _est_body(xflat_ref, alrbd_ref, x_ref, st_ref, w_ref, win_ref, b_ref,
              wout_ref, sr_ref, temp_ref, ns_ref, out_ref):
    u = pl.program_id(0)
    nu = pl.num_programs(0)
    temp = temp_ref[0, 0]

    logits = jnp.dot(xflat_ref[...], alrbd_ref[...],
                     preferred_element_type=jnp.float32) / temp
    lr = jax.nn.softmax(logits, axis=1)
    onehot = (jax.lax.broadcasted_iota(jnp.int32, (1, nu), 1) == u
              ).astype(jnp.float32)
    lr_u = jnp.sum(lr * onehot, axis=1)[:, None]
    sr_u = jnp.sum(sr_ref[...] * onehot)

    x_u = x_ref[...]
    st_u = st_ref[...]
    feed = jnp.dot(x_u, win_ref[0], preferred_element_type=jnp.float32)
    echo = jnp.dot(st_u * sr_u, w_ref[0], preferred_element_type=jnp.float32)
    act = jnp.tanh(feed + echo + b_ref[0, 0, :][None, :])
    ns = (1.0 - lr_u) * st_u + lr_u * act
    ns_ref[...] = ns
    out_ref[...] = jnp.dot(ns, wout_ref[0], preferred_element_type=jnp.float32)


def kernel(X, state, W, Win, bias, Wout, sr, adaptive_lr, temperature,
           w_h, w_o, w_d, win_h, win_o, win_d):
    B, U, D = X.shape
    N = state.shape[2]
    O = Wout.shape[2]
    X_flat = X.reshape(B, U * D)
    st_flat = state.reshape(B, U * N)
    unit_of_row = jnp.repeat(jnp.arange(U), D)
    alr_bd = (adaptive_lr.reshape(U * D)[:, None] *
              (unit_of_row[:, None] == jnp.arange(U)[None, :]))
    sr2 = sr.reshape(1, U)
    temp2 = temperature.reshape(1, 1)
    ns, out = pl.pallas_call(
        _est_body,
        grid=(U,),
        in_specs=[
            pl.BlockSpec((B, U * D), lambda u: (0, 0)),
            pl.BlockSpec((U * D, U), lambda u: (0, 0)),
            pl.BlockSpec((B, D), lambda u: (0, u)),
            pl.BlockSpec((B, N), lambda u: (0, u)),
            pl.BlockSpec((1, N, N), lambda u: (u, 0, 0)),
            pl.BlockSpec((1, D, N), lambda u: (u, 0, 0)),
            pl.BlockSpec((1, 1, N), lambda u: (u, 0, 0)),
            pl.BlockSpec((1, N, O), lambda u: (u, 0, 0)),
            pl.BlockSpec((1, U), lambda u: (0, 0)),
            pl.BlockSpec((1, 1), lambda u: (0, 0)),
        ],
        out_specs=[
            pl.BlockSpec((B, N), lambda u: (0, u)),
            pl.BlockSpec((B, O), lambda u: (0, u)),
        ],
        out_shape=[
            jax.ShapeDtypeStruct((B, U * N), jnp.float32),
            jax.ShapeDtypeStruct((B, U * O), jnp.float32),
        ],
    )(X_flat, alr_bd, X_flat, st_flat, W, Win, bias, Wout, sr2, temp2)
    return ns.reshape(B, U, N), out.reshape(B, U, O)


# speedup vs baseline: 1.0765x; 1.0250x over previous
import jax
import jax.numpy as jnp
from jax.experimental import pallas as pl

_UPG = 2


def _est_body(xflat_ref, alrbd_ref, x_ref, st_ref, w_ref, win_ref, b_ref,
              wout_ref, sr_ref, temp_ref, ns_ref, out_ref):
    g = pl.program_id(0)
    nu = pl.num_programs(0) * _UPG
    temp = temp_ref[0, 0]

    logits = jnp.dot(xflat_ref[...], alrbd_ref[...],
                     preferred_element_type=jnp.float32) / temp
    lr = jax.nn.softmax(logits, axis=1)

    D = x_ref.shape[1] // _UPG
    N = st_ref.shape[1] // _UPG
    for j in range(_UPG):
        u = g * _UPG + j
        onehot = (jax.lax.broadcasted_iota(jnp.int32, (1, nu), 1) == u
                  ).astype(jnp.float32)
        lr_u = jnp.sum(lr * onehot, axis=1)[:, None]
        sr_u = jnp.sum(sr_ref[...] * onehot)
        x_u = x_ref[:, j * D:(j + 1) * D]
        st_u = st_ref[:, j * N:(j + 1) * N]
        feed = jnp.dot(x_u, win_ref[j], preferred_element_type=jnp.float32)
        echo = jnp.dot(st_u * sr_u, w_ref[j],
                       preferred_element_type=jnp.float32)
        act = jnp.tanh(feed + echo + b_ref[j, 0, :][None, :])
        ns = (1.0 - lr_u) * st_u + lr_u * act
        ns_ref[:, j * N:(j + 1) * N] = ns
        out = jnp.dot(ns, wout_ref[j], preferred_element_type=jnp.float32)
        out_ref[:, j * out.shape[1]:(j + 1) * out.shape[1]] = out


def kernel(X, state, W, Win, bias, Wout, sr, adaptive_lr, temperature,
           w_h, w_o, w_d, win_h, win_o, win_d):
    B, U, D = X.shape
    N = state.shape[2]
    O = Wout.shape[2]
    G = U // _UPG
    X_flat = X.reshape(B, U * D)
    st_flat = state.reshape(B, U * N)
    unit_of_row = jnp.repeat(jnp.arange(U), D)
    alr_bd = (adaptive_lr.reshape(U * D)[:, None] *
              (unit_of_row[:, None] == jnp.arange(U)[None, :]))
    sr2 = sr.reshape(1, U)
    temp2 = temperature.reshape(1, 1)
    P = _UPG
    ns, out = pl.pallas_call(
        _est_body,
        grid=(G,),
        in_specs=[
            pl.BlockSpec((B, U * D), lambda g: (0, 0)),
            pl.BlockSpec((U * D, U), lambda g: (0, 0)),
            pl.BlockSpec((B, P * D), lambda g: (0, g)),
            pl.BlockSpec((B, P * N), lambda g: (0, g)),
            pl.BlockSpec((P, N, N), lambda g: (g, 0, 0)),
            pl.BlockSpec((P, D, N), lambda g: (g, 0, 0)),
            pl.BlockSpec((P, 1, N), lambda g: (g, 0, 0)),
            pl.BlockSpec((P, N, O), lambda g: (g, 0, 0)),
            pl.BlockSpec((1, U), lambda g: (0, 0)),
            pl.BlockSpec((1, 1), lambda g: (0, 0)),
        ],
        out_specs=[
            pl.BlockSpec((B, P * N), lambda g: (0, g)),
            pl.BlockSpec((B, P * O), lambda g: (0, g)),
        ],
        out_shape=[
            jax.ShapeDtypeStruct((B, U * N), jnp.float32),
            jax.ShapeDtypeStruct((B, U * O), jnp.float32),
        ],
    )(X_flat, alr_bd, X_flat, st_flat, W, Win, bias, Wout, sr2, temp2)
    return ns.reshape(B, U, N), out.reshape(B, U, O)
